# baseline (speedup 1.0000x reference)
import jax
import jax.numpy as jnp
import numpy as np
from jax.experimental import pallas as pl
from jax.experimental.pallas import tpu as pltpu

_B, _S, _D = 4, 100, 768
_M = 40
_SP = 128
_NCP = 128
_LO = 8
_H = 768


def _fused_kernel(x_ref, hm_ref, sr_ref, W1_ref, b1_ref, W2_ref, b2_ref,
                  Wp1_ref, bp1_ref, Wp2_ref, bp2_ref, out_ref):
    f32 = jnp.float32
    xb = x_ref[0]
    hm_row = hm_ref[0]

    sub = jax.lax.broadcasted_iota(jnp.int32, (_SP, _S), 0).astype(f32)
    ln = jax.lax.broadcasted_iota(jnp.int32, (_SP, _S), 1).astype(f32)
    ident = jnp.where(sub == ln, 1.0, 0.0)
    hm_col = jnp.sum(ident * hm_row, axis=1, keepdims=True)
    gt = jnp.where(hm_row > hm_col, 1.0, 0.0)
    eq = jnp.where((hm_row == hm_col) & (ln < sub), 1.0, 0.0)
    rank = jnp.sum(gt + eq, axis=1, keepdims=True)
    rank = jnp.where(sub[:, 0:1] < float(_S), rank, float(2 * _SP))

    sel_j = jnp.where(sub < float(_M), 1.0, 0.0)
    maskr = jnp.sum(jnp.where(rank == ln, 1.0, 0.0) * sel_j,
                    axis=0, keepdims=True)
    mask_col = jnp.sum(ident * maskr, axis=1, keepdims=True)
    c = jnp.sum(jnp.where(sub <= ln, 1.0, 0.0) * mask_col,
                axis=0, keepdims=True)
    P = maskr * jnp.where((c - 1.0) == sub, 1.0, 0.0)

    hi = jax.lax.Precision.HIGHEST
    xr = jnp.dot(P, xb, preferred_element_type=f32, precision=hi)[0:_M]
    g2 = jnp.dot(P, sr_ref[...], preferred_element_type=f32, precision=hi)[0:_M]
    ln2 = jax.lax.broadcasted_iota(jnp.int32, (2, _LO), 1)
    sub2 = jax.lax.broadcasted_iota(jnp.int32, (2, _LO), 0)
    EA = jnp.where(ln2 == sub2 + 4, 1.0, 0.0)
    EB = jnp.where(ln2 == sub2 + 6, 1.0, 0.0)
    srA = jnp.dot(g2, EA, preferred_element_type=f32, precision=hi)
    srB = jnp.dot(g2, EB, preferred_element_type=f32, precision=hi)

    ln4 = jax.lax.broadcasted_iota(jnp.int32, (4, _LO), 1)
    sub4 = jax.lax.broadcasted_iota(jnp.int32, (4, _LO), 0)
    E4 = jnp.where(ln4 == sub4, 1.0, 0.0)
    W2p = jnp.dot(W2_ref[...], E4, preferred_element_type=f32, precision=hi)
    b2p = jnp.dot(b2_ref[...], E4, preferred_element_type=f32, precision=hi)
    Wp2p = jnp.dot(Wp2_ref[...], E4, preferred_element_type=f32, precision=hi)
    bp2p = jnp.dot(bp2_ref[...], E4, preferred_element_type=f32, precision=hi)

    s1 = jnp.maximum(jnp.dot(xr, W1_ref[...], preferred_element_type=f32)
                     + b1_ref[...], 0.0)
    hr = jnp.dot(s1, W2p, preferred_element_type=f32) + b2p

    A = jnp.dot(xr, Wp1_ref[0:_D], preferred_element_type=f32)
    Bq = jnp.dot(xr, Wp1_ref[_D:2 * _D], preferred_element_type=f32)
    xprod = (xr[:, None, :] * xr[None, :, :]).reshape(_M * _M, _D)
    C = jnp.dot(xprod, Wp1_ref[2 * _D:3 * _D], preferred_element_type=f32)
    hid = jnp.maximum(C.reshape(_M, _M, _H) + A[:, None, :] + Bq[None, :, :]
                      + bp1_ref[...], 0.0).reshape(_M * _M, _H)
    outp = jnp.dot(hid, Wp2p, preferred_element_type=f32) + bp2p
    packed = (outp.reshape(_M, _M, _LO) + hr[None, :, :] + hr[:, None, :]
              + srA[:, None, :] + srB[None, :, :])
    out_ref[0] = packed.reshape(_M * _M, _LO)


def kernel(x, span_ranges, W1, b1, W2, b2, Wp1, bp1, Wp2, bp2):
    f32 = jnp.float32
    B, S, D = x.shape
    NC = W2.shape[1]

    h = jax.nn.relu(x @ W1 + b1) @ W2 + b2
    hm = jnp.mean(h[:, :, 1:4], axis=-1)

    full = lambda shp: pl.BlockSpec(shp, lambda b: (0,) * len(shp))
    out = pl.pallas_call(
        _fused_kernel,
        grid=(B,),
        in_specs=[
            pl.BlockSpec((1, S, D), lambda b: (b, 0, 0)),
            pl.BlockSpec((1, 1, S), lambda b: (b, 0, 0)),
            full((S, 2)),
            full((D, _H)), full((1, _H)), full((D, NC)), full((1, NC)),
            full((3 * D, _H)), full((1, _H)), full((_H, NC)), full((1, NC)),
        ],
        out_specs=pl.BlockSpec((1, _M * _M, _LO), lambda b: (b, 0, 0)),
        out_shape=jax.ShapeDtypeStruct((B, _M * _M, _LO), f32),
    )(x, hm[:, None, :], span_ranges.astype(f32), W1, b1.reshape(1, _H),
      W2, b2.reshape(1, NC), Wp1, bp1.reshape(1, _H), Wp2, bp2.reshape(1, NC))

    summed = out[:, :, :NC]
    span_pair_ranges = jnp.round(out[:, :, 4:8]).astype(jnp.int32).reshape(B, _M * _M, 2, 2)
    return summed, span_pair_ranges


# speedup vs baseline: 1.0027x; 1.0027x over previous
import jax
import jax.numpy as jnp
import numpy as np
from jax.experimental import pallas as pl
from jax.experimental.pallas import tpu as pltpu

_B, _S, _D = 4, 100, 768
_M = 40
_SP = 128
_NCP = 128
_H = 768


def _fused_kernel(x_ref, hm_ref, sr_ref, W1_ref, b1_ref, W2_ref, b2_ref,
                  Wp1_ref, bp1_ref, Wp2_ref, bp2_ref, out_ref):
    f32 = jnp.float32
    xb = x_ref[0]
    hm_row = hm_ref[0]

    sub = jax.lax.broadcasted_iota(jnp.int32, (_SP, _S), 0).astype(f32)
    ln = jax.lax.broadcasted_iota(jnp.int32, (_SP, _S), 1).astype(f32)
    ident = jnp.where(sub == ln, 1.0, 0.0)
    hm_col = jnp.sum(ident * hm_row, axis=1, keepdims=True)
    gt = jnp.where(hm_row > hm_col, 1.0, 0.0)
    eq = jnp.where((hm_row == hm_col) & (ln < sub), 1.0, 0.0)
    rank = jnp.sum(gt + eq, axis=1, keepdims=True)
    rank = jnp.where(sub[:, 0:1] < float(_S), rank, float(2 * _SP))

    sel_j = jnp.where(sub < float(_M), 1.0, 0.0)
    maskr = jnp.sum(jnp.where(rank == ln, 1.0, 0.0) * sel_j,
                    axis=0, keepdims=True)
    mask_col = jnp.sum(ident * maskr, axis=1, keepdims=True)
    c = jnp.sum(jnp.where(sub <= ln, 1.0, 0.0) * mask_col,
                axis=0, keepdims=True)
    P = maskr * jnp.where((c - 1.0) == sub, 1.0, 0.0)

    hi = jax.lax.Precision.HIGHEST
    xr = jnp.dot(P, xb, preferred_element_type=f32, precision=hi)[0:_M]
    g2 = jnp.dot(P, sr_ref[...], preferred_element_type=f32, precision=hi)[0:_M]
    ln2 = jax.lax.broadcasted_iota(jnp.int32, (2, _NCP), 1)
    sub2 = jax.lax.broadcasted_iota(jnp.int32, (2, _NCP), 0)
    EA = jnp.where(ln2 == sub2 + 4, 1.0, 0.0)
    EB = jnp.where(ln2 == sub2 + 6, 1.0, 0.0)
    srA = jnp.dot(g2, EA, preferred_element_type=f32, precision=hi)
    srB = jnp.dot(g2, EB, preferred_element_type=f32, precision=hi)

    ln4 = jax.lax.broadcasted_iota(jnp.int32, (4, _NCP), 1)
    sub4 = jax.lax.broadcasted_iota(jnp.int32, (4, _NCP), 0)
    E4 = jnp.where(ln4 == sub4, 1.0, 0.0)
    W2p = jnp.dot(W2_ref[...], E4, preferred_element_type=f32, precision=hi)
    b2p = jnp.dot(b2_ref[...], E4, preferred_element_type=f32, precision=hi)
    Wp2p = jnp.dot(Wp2_ref[...], E4, preferred_element_type=f32, precision=hi)
    bp2p = jnp.dot(bp2_ref[...], E4, preferred_element_type=f32, precision=hi)

    s1 = jnp.maximum(jnp.dot(xr, W1_ref[...], preferred_element_type=f32)
                     + b1_ref[...], 0.0)
    hr = jnp.dot(s1, W2p, preferred_element_type=f32) + b2p

    A = jnp.dot(xr, Wp1_ref[0:_D], preferred_element_type=f32)
    Bq = jnp.dot(xr, Wp1_ref[_D:2 * _D], preferred_element_type=f32)
    xprod = (xr[:, None, :] * xr[None, :, :]).reshape(_M * _M, _D)
    C = jnp.dot(xprod, Wp1_ref[2 * _D:3 * _D], preferred_element_type=f32)
    hid = jnp.maximum(C.reshape(_M, _M, _H) + A[:, None, :] + Bq[None, :, :]
                      + bp1_ref[...], 0.0).reshape(_M * _M, _H)
    outp = jnp.dot(hid, Wp2p, preferred_element_type=f32) + bp2p
    packed = (outp.reshape(_M, _M, _NCP) + hr[None, :, :] + hr[:, None, :]
              + srA[:, None, :] + srB[None, :, :])
    out_ref[0] = packed.reshape(_M * _M, _NCP)


def kernel(x, span_ranges, W1, b1, W2, b2, Wp1, bp1, Wp2, bp2):
    f32 = jnp.float32
    B, S, D = x.shape
    NC = W2.shape[1]

    h = jax.nn.relu(x @ W1 + b1) @ W2 + b2
    hm = jnp.mean(h[:, :, 1:4], axis=-1)

    full = lambda shp: pl.BlockSpec(shp, lambda b: (0,) * len(shp))
    out = pl.pallas_call(
        _fused_kernel,
        grid=(B,),
        in_specs=[
            pl.BlockSpec((1, S, D), lambda b: (b, 0, 0)),
            pl.BlockSpec((1, 1, S), lambda b: (b, 0, 0)),
            full((S, 2)),
            full((D, _H)), full((1, _H)), full((D, NC)), full((1, NC)),
            full((3 * D, _H)), full((1, _H)), full((_H, NC)), full((1, NC)),
        ],
        out_specs=pl.BlockSpec((1, _M * _M, _NCP), lambda b: (b, 0, 0)),
        out_shape=jax.ShapeDtypeStruct((B, _M * _M, _NCP), f32),
    )(x, hm[:, None, :], span_ranges.astype(f32), W1, b1.reshape(1, _H),
      W2, b2.reshape(1, NC), Wp1, bp1.reshape(1, _H), Wp2, bp2.reshape(1, NC))

    summed = out[:, :, :NC]
    span_pair_ranges = jnp.round(out[:, :, 4:8]).astype(jnp.int32).reshape(B, _M * _M, 2, 2)
    return summed, span_pair_ranges


# speedup vs baseline: 1.0878x; 1.0849x over previous
import jax
import jax.numpy as jnp
import numpy as np
from jax.experimental import pallas as pl
from jax.experimental.pallas import tpu as pltpu

_B, _S, _D = 4, 100, 768
_M = 40
_SP = 128
_NCP = 128
_H = 768


def _fused_kernel(x_ref, hm_ref, sr_ref, W1_ref, b1_ref, W2_ref, b2_ref,
                  Wp1_ref, bp1_ref, Wp2_ref, bp2_ref, out_ref, spr_ref):
    f32 = jnp.float32
    xb = x_ref[0]
    hm_row = hm_ref[0]

    sub = jax.lax.broadcasted_iota(jnp.int32, (_SP, _S), 0).astype(f32)
    ln = jax.lax.broadcasted_iota(jnp.int32, (_SP, _S), 1).astype(f32)
    ident = jnp.where(sub == ln, 1.0, 0.0)
    hm_col = jnp.sum(ident * hm_row, axis=1, keepdims=True)
    gt = jnp.where(hm_row > hm_col, 1.0, 0.0)
    eq = jnp.where((hm_row == hm_col) & (ln < sub), 1.0, 0.0)
    rank = jnp.sum(gt + eq, axis=1, keepdims=True)
    rank = jnp.where(sub[:, 0:1] < float(_S), rank, float(2 * _SP))

    sel_j = jnp.where(sub < float(_M), 1.0, 0.0)
    maskr = jnp.sum(jnp.where(rank == ln, 1.0, 0.0) * sel_j,
                    axis=0, keepdims=True)
    mask_col = jnp.sum(ident * maskr, axis=1, keepdims=True)
    c = jnp.sum(jnp.where(sub <= ln, 1.0, 0.0) * mask_col,
                axis=0, keepdims=True)
    P = maskr * jnp.where((c - 1.0) == sub, 1.0, 0.0)

    hi = jax.lax.Precision.HIGHEST
    xr = jnp.dot(P, xb, preferred_element_type=f32, precision=hi)[0:_M]
    g2 = jnp.dot(P, sr_ref[...].astype(f32), preferred_element_type=f32,
                 precision=hi)[0:_M]
    ln2 = jax.lax.broadcasted_iota(jnp.int32, (2, _NCP), 1)
    sub2 = jax.lax.broadcasted_iota(jnp.int32, (2, _NCP), 0)
    EA = jnp.where(ln2 == sub2, 1.0, 0.0)
    EB = jnp.where(ln2 == sub2 + 2, 1.0, 0.0)
    srA = jnp.dot(g2, EA, preferred_element_type=f32, precision=hi)
    srB = jnp.dot(g2, EB, preferred_element_type=f32, precision=hi)

    ln4 = jax.lax.broadcasted_iota(jnp.int32, (4, _NCP), 1)
    sub4 = jax.lax.broadcasted_iota(jnp.int32, (4, _NCP), 0)
    E4 = jnp.where(ln4 == sub4, 1.0, 0.0)
    W2p = jnp.dot(W2_ref[...], E4, preferred_element_type=f32, precision=hi)
    b2p = jnp.dot(b2_ref[...], E4, preferred_element_type=f32, precision=hi)
    Wp2p = jnp.dot(Wp2_ref[...], E4, preferred_element_type=f32, precision=hi)
    bp2p = jnp.dot(bp2_ref[...], E4, preferred_element_type=f32, precision=hi)

    s1 = jnp.maximum(jnp.dot(xr, W1_ref[...], preferred_element_type=f32)
                     + b1_ref[...], 0.0)
    hr = jnp.dot(s1, W2p, preferred_element_type=f32) + b2p

    A = jnp.dot(xr, Wp1_ref[0:_D], preferred_element_type=f32)
    Bq = jnp.dot(xr, Wp1_ref[_D:2 * _D], preferred_element_type=f32)
    xprod = (xr[:, None, :] * xr[None, :, :]).reshape(_M * _M, _D)
    C = jnp.dot(xprod, Wp1_ref[2 * _D:3 * _D], preferred_element_type=f32)
    hid = jnp.maximum(C.reshape(_M, _M, _H) + A[:, None, :] + Bq[None, :, :]
                      + bp1_ref[...], 0.0).reshape(_M * _M, _H)
    outp = jnp.dot(hid, Wp2p, preferred_element_type=f32) + bp2p
    summed = outp.reshape(_M, _M, _NCP) + hr[None, :, :] + hr[:, None, :]
    out_ref[0] = summed.reshape(_M * _M, _NCP)

    spr = srA[:, None, :] + srB[None, :, :]
    spr_ref[0] = spr.reshape(_M * _M, _NCP)


def kernel(x, span_ranges, W1, b1, W2, b2, Wp1, bp1, Wp2, bp2):
    f32 = jnp.float32
    B, S, D = x.shape
    NC = W2.shape[1]

    h = jax.nn.relu(x @ W1 + b1) @ W2 + b2
    hm = jnp.mean(h[:, :, 1:4], axis=-1)

    full = lambda shp: pl.BlockSpec(shp, lambda b: (0,) * len(shp))
    out, spr = pl.pallas_call(
        _fused_kernel,
        grid=(B,),
        in_specs=[
            pl.BlockSpec((1, S, D), lambda b: (b, 0, 0)),
            pl.BlockSpec((1, 1, S), lambda b: (b, 0, 0)),
            full((S, 2)),
            full((D, _H)), full((1, _H)), full((D, NC)), full((1, NC)),
            full((3 * D, _H)), full((1, _H)), full((_H, NC)), full((1, NC)),
        ],
        out_specs=[
            pl.BlockSpec((1, _M * _M, _NCP), lambda b: (b, 0, 0)),
            pl.BlockSpec((1, _M * _M, _NCP), lambda b: (b, 0, 0)),
        ],
        out_shape=[
            jax.ShapeDtypeStruct((B, _M * _M, _NCP), f32),
            jax.ShapeDtypeStruct((B, _M * _M, _NCP), f32),
        ],
    )(x, hm[:, None, :], span_ranges, W1, b1.reshape(1, _H),
      W2, b2.reshape(1, NC), Wp1, bp1.reshape(1, _H), Wp2, bp2.reshape(1, NC))

    summed = out[:, :, :NC]
    span_pair_ranges = jnp.round(spr[:, :, :4]).astype(jnp.int32).reshape(B, _M * _M, 2, 2)
    return summed, span_pair_ranges
